# baseline (speedup 1.0000x reference)
import functools

import jax
import jax.numpy as jnp
from jax import lax
from jax.experimental import pallas as pl
from jax.experimental.pallas import tpu as pltpu
from jax.experimental.pallas import tpu_sc as plsc

B, C, L, D = 64, 8, 576, 192
KEEP = 144
NW = 32
B_PER_W = B // NW
LCH = L // 16
KCH = KEEP // 16


RANK_BLK = 8


def _rank_body(noise_ref, rank_ref):
    li = lax.broadcasted_iota(jnp.int32, (L, L), 0)
    ji = lax.broadcasted_iota(jnp.int32, (L, L), 1)
    tie = ji < li
    for i in range(RANK_BLK):
        row = noise_ref[i, :]
        a = row[:, None]
        bt = row[None, :]
        cmp = (bt < a) | ((bt == a) & tie)
        rank_ref[i, :] = jnp.sum(cmp.astype(jnp.int32), axis=1)


def _compute_ranks(noise):
    return pl.pallas_call(
        _rank_body,
        grid=(B // RANK_BLK,),
        in_specs=[pl.BlockSpec((RANK_BLK, L), lambda b: (b, 0))],
        out_specs=pl.BlockSpec((RANK_BLK, L), lambda b: (b, 0)),
        out_shape=jax.ShapeDtypeStruct((B, L), jnp.int32),
    )(noise)


def _sc_body(x_ref, rank_ref, xk_ref, idr_ref, mask_ref, idk_ref,
             rank_row, keep, mrow, gidx, rows, sem):
    cid = lax.axis_index("c")
    sid = lax.axis_index("s")
    wid = sid * 2 + cid
    for i in range(B_PER_W):
        b = wid * B_PER_W + i
        pltpu.sync_copy(rank_ref.at[pl.ds(b * L, L)], rank_row)
        for k in range(LCH):
            r = rank_row[pl.ds(k * 16, 16)]
            lvec = lax.iota(jnp.int32, 16) + (k * 16)
            m = r < KEEP
            idx = jnp.where(m, r, 0)
            plsc.store_scatter(keep, [idx], lvec, mask=m)
            mrow[pl.ds(k * 16, 16)] = jnp.where(
                m, jnp.float32(0.0), jnp.float32(1.0))
        for c in range(C):
            bc = b * C + c
            pltpu.sync_copy(rank_row, idr_ref.at[pl.ds(bc * L, L)])
            pltpu.sync_copy(mrow, mask_ref.at[pl.ds(bc * L, L)])
            pltpu.sync_copy(keep, idk_ref.at[pl.ds(bc * KEEP, KEEP)])
            base = bc * L
            for k in range(KCH):
                gidx[pl.ds(k * 16, 16)] = keep[pl.ds(k * 16, 16)] + base
            c1 = pltpu.async_copy(
                x_ref.at[gidx.at[pl.ds(0, 80)]], rows.at[pl.ds(0, 80)], sem)
            c2 = pltpu.async_copy(
                x_ref.at[gidx.at[pl.ds(80, 64)]], rows.at[pl.ds(80, 64)], sem)
            c1.wait()
            c2.wait()
            pltpu.sync_copy(rows, xk_ref.at[pl.ds(bc * KEEP, KEEP)])


@functools.cache
def _sc_dropout():
    return pl.kernel(
        _sc_body,
        out_type=(
            jax.ShapeDtypeStruct((B * C * KEEP, D), jnp.float32),
            jax.ShapeDtypeStruct((B * C * L,), jnp.int32),
            jax.ShapeDtypeStruct((B * C * L,), jnp.float32),
            jax.ShapeDtypeStruct((B * C * KEEP,), jnp.int32),
        ),
        mesh=plsc.VectorSubcoreMesh(core_axis_name="c", subcore_axis_name="s"),
        scratch_types=[
            pltpu.VMEM((L,), jnp.int32),
            pltpu.VMEM((KEEP,), jnp.int32),
            pltpu.VMEM((L,), jnp.float32),
            pltpu.VMEM((KEEP,), jnp.int32),
            pltpu.VMEM((KEEP, D), jnp.float32),
            pltpu.SemaphoreType.DMA,
        ],
        compiler_params=pltpu.CompilerParams(needs_layout_passes=False,
                                             use_tc_tiling_on_sc=False),
    )


def kernel(x):
    assert x.shape == (B, C, L, D), x.shape
    noise = jax.random.uniform(jax.random.key(1), (B, L), dtype=jnp.float32)
    rank = _compute_ranks(noise)
    xk, idr, mask, idk = _sc_dropout()(
        x.reshape(B * C * L, D), rank.reshape(B * L))
    return (xk.reshape(B, C, KEEP, D), idr.reshape(B, C, L),
            mask.reshape(B, C, L), idk.reshape(B, C, KEEP))
